# Initial kernel scaffold; baseline (speedup 1.0000x reference)
import jax
import jax.numpy as jnp
from jax.experimental import pallas as pl


def kernel(x, edge_index, edge_attr, global_attr, W, b):
    raise NotImplementedError("write your pallas kernel here")


# speedup vs baseline: 4.9707x; 4.9707x over previous
import functools

import jax
import jax.numpy as jnp
from jax import lax
from jax.experimental import pallas as pl
from jax.experimental.pallas import tpu as pltpu
from jax.experimental.pallas import tpu_sc as plsc

N = 10000
E = 160000
DF = 256
DE = 16
DG = 128
NC = 2
NS = 16
NW = NC * NS
EPW = E // NW
CH = 128
NCH = (EPW + CH - 1) // CH
EPW_PAD = NCH * CH
N_PAD = 10112
RPT = N_PAD // NS


def _sc_segment_sum(dst_p, edge_attr):
  mesh = plsc.VectorSubcoreMesh(core_axis_name="c", subcore_axis_name="s")

  @functools.partial(
      pl.kernel,
      out_type=[
          jax.ShapeDtypeStruct((NC, N_PAD, DE), jnp.float32),
          jax.ShapeDtypeStruct((NC, N_PAD, DE), jnp.float32),
      ],
      mesh=mesh,
      compiler_params=pltpu.CompilerParams(use_tc_tiling_on_sc=False),
      scratch_types=[
          pltpu.VMEM((NCH, CH), jnp.int32),
          pltpu.VMEM((EPW_PAD, DE), jnp.float32),
          pltpu.VMEM((CH, DE), jnp.float32),
          pltpu.VMEM((RPT, DE), jnp.float32),
          pltpu.VMEM_SHARED((N_PAD, DE), jnp.float32),
          pltpu.VMEM_SHARED((N_PAD, DE), jnp.float32),
      ],
  )
  def seg_kernel(dst_hbm, attr_hbm, sums_out, cnt_out,
                 idx_v, vals_v, ones_v, zeros_v, sums_sh, cnt_sh):
    c = lax.axis_index("c")
    s = lax.axis_index("s")
    wid = c * NS + s

    one_row = jnp.ones((DE,), jnp.float32)
    zero_row = jnp.zeros((DE,), jnp.float32)

    def fill_ones(i, carry):
      ones_v[i, :] = one_row
      return carry
    lax.fori_loop(0, CH, fill_ones, 0)

    def fill_zeros(i, carry):
      zeros_v[i, :] = zero_row
      return carry
    lax.fori_loop(0, RPT, fill_zeros, 0)

    row0 = pl.multiple_of(s * RPT, RPT)
    pltpu.sync_copy(zeros_v, sums_sh.at[pl.ds(row0, RPT)])
    pltpu.sync_copy(zeros_v, cnt_sh.at[pl.ds(row0, RPT)])

    pltpu.sync_copy(dst_hbm.at[wid], idx_v)
    ebase = pl.multiple_of(wid * EPW, EPW)
    pltpu.sync_copy(attr_hbm.at[pl.ds(ebase, EPW)], vals_v.at[pl.ds(0, EPW)])

    plsc.subcore_barrier()

    def chunk(j, carry):
      off = pl.multiple_of(j * CH, CH)
      pltpu.sync_copy(vals_v.at[pl.ds(off, CH)], sums_sh.at[idx_v.at[j]],
                      add=True)
      pltpu.sync_copy(ones_v, cnt_sh.at[idx_v.at[j]], add=True)
      return carry
    lax.fori_loop(0, NCH, chunk, 0)

    plsc.subcore_barrier()

    pltpu.sync_copy(sums_sh.at[pl.ds(row0, RPT)],
                    sums_out.at[c, pl.ds(row0, RPT)])
    pltpu.sync_copy(cnt_sh.at[pl.ds(row0, RPT)],
                    cnt_out.at[c, pl.ds(row0, RPT)])

  return seg_kernel(dst_p, edge_attr)


_BN = 1000


def _tc_body(sums_ref, cnt_ref, x_ref, w_ref, g_ref, b_ref, o_ref):
  s = sums_ref[0] + sums_ref[1]
  cnt = cnt_ref[0, :, 0:1] + cnt_ref[1, :, 0:1]
  agg = s / jnp.maximum(cnt, 1.0)
  w_e = w_ref[0:DE, :]
  w_x = w_ref[DE:DE + DF, :]
  w_g = w_ref[DE + DF:, :]
  acc = jnp.dot(x_ref[...], w_x, preferred_element_type=jnp.float32)
  acc += jnp.dot(agg, w_e, preferred_element_type=jnp.float32)
  acc += jnp.dot(g_ref[...], w_g, preferred_element_type=jnp.float32)
  o_ref[...] = acc + b_ref[...]


def kernel(x, edge_index, edge_attr, global_attr, W, b):
  dst = edge_index[1].astype(jnp.int32)
  pad = jnp.full((NW, EPW_PAD - EPW), N, jnp.int32)
  dst_p = jnp.concatenate([dst.reshape(NW, EPW), pad], axis=1)
  dst_p = dst_p.reshape(NW, NCH, CH)

  sums, cnt = _sc_segment_sum(dst_p, edge_attr)
  sums = sums[:, :N, :]
  cnt = cnt[:, :N, :]

  g2 = global_attr.reshape(1, DG)
  b2 = b.reshape(1, DF)

  out = pl.pallas_call(
      _tc_body,
      grid=(N // _BN,),
      in_specs=[
          pl.BlockSpec((NC, _BN, DE), lambda i: (0, i, 0)),
          pl.BlockSpec((NC, _BN, DE), lambda i: (0, i, 0)),
          pl.BlockSpec((_BN, DF), lambda i: (i, 0)),
          pl.BlockSpec((DE + DF + DG, DF), lambda i: (0, 0)),
          pl.BlockSpec((1, DG), lambda i: (0, 0)),
          pl.BlockSpec((1, DF), lambda i: (0, 0)),
      ],
      out_specs=pl.BlockSpec((_BN, DF), lambda i: (i, 0)),
      out_shape=jax.ShapeDtypeStruct((N, DF), jnp.float32),
  )(sums, cnt, x, W, g2, b2)
  return out


# speedup vs baseline: 5.4148x; 1.0893x over previous
import functools

import jax
import jax.numpy as jnp
from jax import lax
from jax.experimental import pallas as pl
from jax.experimental.pallas import tpu as pltpu
from jax.experimental.pallas import tpu_sc as plsc

N = 10000
E = 160000
DF = 256
DE = 16
DG = 128
NC = 2
NS = 16
NW = NC * NS
EPW = E // NW
CH = 128
NCH = (EPW + CH - 1) // CH
EPW_PAD = NCH * CH
N_PAD = 10112
RPT = N_PAD // NS


def _sc_segment_sum(dst_p, edge_attr):
  mesh = plsc.VectorSubcoreMesh(core_axis_name="c", subcore_axis_name="s")

  @functools.partial(
      pl.kernel,
      out_type=[
          jax.ShapeDtypeStruct((NC, N, DE), jnp.float32),
          jax.ShapeDtypeStruct((NC, N, DE), jnp.float32),
      ],
      mesh=mesh,
      compiler_params=pltpu.CompilerParams(use_tc_tiling_on_sc=False),
      scratch_types=[
          pltpu.VMEM((NCH, CH), jnp.int32),
          pltpu.VMEM((EPW_PAD, DE), jnp.float32),
          pltpu.VMEM((CH, DE), jnp.float32),
          pltpu.VMEM((RPT, DE), jnp.float32),
          pltpu.VMEM_SHARED((N_PAD, DE), jnp.float32),
          pltpu.VMEM_SHARED((N_PAD, DE), jnp.float32),
      ],
  )
  def seg_kernel(dst_hbm, attr_hbm, sums_out, cnt_out,
                 idx_v, vals_v, ones_v, zeros_v, sums_sh, cnt_sh):
    c = lax.axis_index("c")
    s = lax.axis_index("s")
    wid = c * NS + s

    one_row = jnp.ones((DE,), jnp.float32)
    zero_row = jnp.zeros((DE,), jnp.float32)

    def fill_ones(i, carry):
      ones_v[i, :] = one_row
      return carry
    lax.fori_loop(0, CH, fill_ones, 0)

    def fill_zeros(i, carry):
      zeros_v[i, :] = zero_row
      return carry
    lax.fori_loop(0, RPT, fill_zeros, 0)

    row0 = pl.multiple_of(s * RPT, RPT)
    pltpu.sync_copy(zeros_v, sums_sh.at[pl.ds(row0, RPT)])
    pltpu.sync_copy(zeros_v, cnt_sh.at[pl.ds(row0, RPT)])

    pltpu.sync_copy(dst_hbm.at[wid], idx_v)
    ebase = pl.multiple_of(wid * EPW, EPW)
    pltpu.sync_copy(attr_hbm.at[pl.ds(ebase, EPW)], vals_v.at[pl.ds(0, EPW)])

    plsc.subcore_barrier()

    def chunk(j, carry):
      off = pl.multiple_of(j * CH, CH)
      pltpu.sync_copy(vals_v.at[pl.ds(off, CH)], sums_sh.at[idx_v.at[j]],
                      add=True)
      pltpu.sync_copy(ones_v, cnt_sh.at[idx_v.at[j]], add=True)
      return carry
    lax.fori_loop(0, NCH, chunk, 0)

    plsc.subcore_barrier()

    @pl.when(s < NS - 1)
    def _full_slab():
      pltpu.sync_copy(sums_sh.at[pl.ds(row0, RPT)],
                      sums_out.at[c, pl.ds(row0, RPT)])
      pltpu.sync_copy(cnt_sh.at[pl.ds(row0, RPT)],
                      cnt_out.at[c, pl.ds(row0, RPT)])

    @pl.when(s == NS - 1)
    def _last_slab():
      last0 = (NS - 1) * RPT
      pltpu.sync_copy(sums_sh.at[pl.ds(last0, N - last0)],
                      sums_out.at[c, pl.ds(last0, N - last0)])
      pltpu.sync_copy(cnt_sh.at[pl.ds(last0, N - last0)],
                      cnt_out.at[c, pl.ds(last0, N - last0)])

  return seg_kernel(dst_p, edge_attr)


_BN = 1000


def _tc_body(sums_ref, cnt_ref, x_ref, w_ref, g_ref, b_ref, o_ref):
  s = sums_ref[0] + sums_ref[1]
  cnt = cnt_ref[0, :, 0:1] + cnt_ref[1, :, 0:1]
  agg = s / jnp.maximum(cnt, 1.0)
  w_e = w_ref[0:DE, :]
  w_x = w_ref[DE:DE + DF, :]
  w_g = w_ref[DE + DF:, :]
  acc = jnp.dot(x_ref[...], w_x, preferred_element_type=jnp.float32)
  acc += jnp.dot(agg, w_e, preferred_element_type=jnp.float32)
  acc += jnp.dot(g_ref[...], w_g, preferred_element_type=jnp.float32)
  o_ref[...] = acc + b_ref[...]


def kernel(x, edge_index, edge_attr, global_attr, W, b):
  dst = edge_index[1].astype(jnp.int32)
  pad = jnp.full((NW, EPW_PAD - EPW), N, jnp.int32)
  dst_p = jnp.concatenate([dst.reshape(NW, EPW), pad], axis=1)
  dst_p = dst_p.reshape(NW, NCH, CH)

  sums, cnt = _sc_segment_sum(dst_p, edge_attr)

  g2 = global_attr.reshape(1, DG)
  b2 = b.reshape(1, DF)

  out = pl.pallas_call(
      _tc_body,
      grid=(N // _BN,),
      in_specs=[
          pl.BlockSpec((NC, _BN, DE), lambda i: (0, i, 0)),
          pl.BlockSpec((NC, _BN, DE), lambda i: (0, i, 0)),
          pl.BlockSpec((_BN, DF), lambda i: (i, 0)),
          pl.BlockSpec((DE + DF + DG, DF), lambda i: (0, 0)),
          pl.BlockSpec((1, DG), lambda i: (0, 0)),
          pl.BlockSpec((1, DF), lambda i: (0, 0)),
      ],
      out_specs=pl.BlockSpec((_BN, DF), lambda i: (i, 0)),
      out_shape=jax.ShapeDtypeStruct((N, DF), jnp.float32),
  )(sums, cnt, x, W, g2, b2)
  return out
